# Initial kernel scaffold; baseline (speedup 1.0000x reference)
import jax
import jax.numpy as jnp
from jax.experimental import pallas as pl


def kernel(mb_t, mtd, W):
    raise NotImplementedError("write your pallas kernel here")


# speedup vs baseline: 12.6101x; 12.6101x over previous
import functools

import jax
import jax.numpy as jnp
from jax import lax
from jax.experimental import pallas as pl
from jax.experimental.pallas import tpu as pltpu
from jax.experimental.pallas import tpu_sc as plsc

VOCAB = 1000000
EMB = 32
B = 4096
S = 50
C = 20

NSEG = B * S
NW = 32
SEG_PER_W = NSEG // NW
G = 64
CHUNKS = SEG_PER_W // G
IDX_PER_CHUNK = G * C
NSUB = IDX_PER_CHUNK // 128


def _make_kernel():
    mesh = plsc.VectorSubcoreMesh(core_axis_name="c", subcore_axis_name="s")

    @functools.partial(
        pl.kernel,
        mesh=mesh,
        out_type=jax.ShapeDtypeStruct((NSEG, EMB), jnp.float32),
        compiler_params=pltpu.CompilerParams(use_tc_tiling_on_sc=False),
        scratch_types=[
            pltpu.VMEM((IDX_PER_CHUNK,), jnp.int32),
            pltpu.VMEM((IDX_PER_CHUNK, EMB), jnp.float32),
            pltpu.VMEM((G, EMB), jnp.float32),
            pltpu.SemaphoreType.DMA,
        ],
    )
    def k(idx_hbm, table_hbm, out_hbm, idx_v, rows_v, out_v, sem):
        wid = lax.axis_index("s") * 2 + lax.axis_index("c")
        chunk0 = wid * CHUNKS

        def chunk_body(g, _):
            chunk = chunk0 + g
            pltpu.sync_copy(idx_hbm.at[pl.ds(chunk * IDX_PER_CHUNK, IDX_PER_CHUNK)], idx_v)
            handles = []
            for j in range(NSUB):
                handles.append(
                    pltpu.async_copy(
                        table_hbm.at[idx_v.at[pl.ds(j * 128, 128)]],
                        rows_v.at[pl.ds(j * 128, 128)],
                        sem,
                    )
                )
            for h in handles:
                h.wait()

            def seg_body(i, _):
                r0 = i * C
                acc0 = rows_v[r0, pl.ds(0, 16)]
                acc1 = rows_v[r0, pl.ds(16, 16)]
                for c in range(1, C):
                    acc0 = acc0 + rows_v[r0 + c, pl.ds(0, 16)]
                    acc1 = acc1 + rows_v[r0 + c, pl.ds(16, 16)]
                out_v[i, pl.ds(0, 16)] = acc0
                out_v[i, pl.ds(16, 16)] = acc1
                return 0

            lax.fori_loop(0, G, seg_body, 0)
            pltpu.sync_copy(out_v, out_hbm.at[pl.ds(chunk * G, G)])
            return 0

        lax.fori_loop(0, CHUNKS, chunk_body, 0)

    return k


_sc_kernel = _make_kernel()


def kernel(mb_t, mtd, W):
    del mtd
    idx = mb_t.astype(jnp.int32).reshape(NSEG * C)
    out = _sc_kernel(idx, W)
    return out.reshape(B, S, EMB)


# speedup vs baseline: 15.3278x; 1.2155x over previous
import functools

import jax
import jax.numpy as jnp
from jax import lax
from jax.experimental import pallas as pl
from jax.experimental.pallas import tpu as pltpu
from jax.experimental.pallas import tpu_sc as plsc

VOCAB = 1000000
EMB = 32
B = 4096
S = 50
C = 20

NSEG = B * S
NW = 32
G = 64
UNITS = NSEG // G
UNITS_PER_W = UNITS // NW


def _tree_sum(vals):
    while len(vals) > 1:
        nxt = [vals[i] + vals[i + 1] for i in range(0, len(vals) - 1, 2)]
        if len(vals) % 2:
            nxt.append(vals[-1])
        vals = nxt
    return vals[0]


def _make_kernel():
    mesh = plsc.VectorSubcoreMesh(core_axis_name="c", subcore_axis_name="s")

    @functools.partial(
        pl.kernel,
        mesh=mesh,
        out_type=jax.ShapeDtypeStruct((S, EMB // 8, B // 128, 8, 128), jnp.float32),
        compiler_params=pltpu.CompilerParams(
            use_tc_tiling_on_sc=False, needs_layout_passes=False
        ),
        scratch_types=[
            pltpu.VMEM((C, G), jnp.int32),
            pltpu.VMEM((C, G, EMB), jnp.float32),
            pltpu.VMEM((EMB // 8, 8, G), jnp.float32),
            pltpu.SemaphoreType.DMA,
        ],
    )
    def k(idx_hbm, table_hbm, out_hbm, idx_v, rows_v, out_v, sem):
        wid = lax.axis_index("s") * 2 + lax.axis_index("c")
        u0 = wid * UNITS_PER_W
        lane = lax.iota(jnp.int32, 16)
        e_hi0 = lane >> 3
        e_lo0 = lane & 7
        e_hi1 = e_hi0 + 2

        def chunk_body(j, _):
            u = u0 + j
            s0 = u * G
            pltpu.sync_copy(idx_hbm.at[:, pl.ds(s0, G)], idx_v)
            handles = []
            for c in range(C):
                handles.append(
                    pltpu.async_copy(
                        table_hbm.at[idx_v.at[c]], rows_v.at[c], sem
                    )
                )
            for h in handles:
                h.wait()

            def seg_body(i, _):
                acc0 = _tree_sum([rows_v[c, i, pl.ds(0, 16)] for c in range(C)])
                acc1 = _tree_sum([rows_v[c, i, pl.ds(16, 16)] for c in range(C)])
                bvec = jnp.full((16,), i, jnp.int32)
                plsc.store_scatter(out_v, [e_hi0, e_lo0, bvec], acc0)
                plsc.store_scatter(out_v, [e_hi1, e_lo0, bvec], acc1)
                return 0

            lax.fori_loop(0, G, seg_body, 0)
            sv = s0 // B
            rem = s0 - sv * B
            bt = rem // 128
            half = rem - bt * 128
            pltpu.sync_copy(
                out_v, out_hbm.at[sv, :, bt, :, pl.ds(half, G)]
            )
            return 0

        lax.fori_loop(0, UNITS_PER_W, chunk_body, 0)

    return k


_sc_kernel = _make_kernel()


def kernel(mb_t, mtd, W):
    del mtd
    idx = mb_t.astype(jnp.int32).transpose(2, 1, 0).reshape(C, S * B)
    out5 = _sc_kernel(idx, W)
    return out5.transpose(2, 4, 0, 1, 3).reshape(B, S, EMB)
